# Initial kernel scaffold; baseline (speedup 1.0000x reference)
import jax
import jax.numpy as jnp
from jax.experimental import pallas as pl


def kernel(xyz1, xyz2, points2, W0, W1, wn_w0, wn_b0, wn_w1, wn_b1, wn_w2, wn_b2, W_lin):
    raise NotImplementedError("write your pallas kernel here")


# speedup vs baseline: 1.1276x; 1.1276x over previous
import functools

import jax
import jax.numpy as jnp
from jax.experimental import pallas as pl
from jax.experimental.pallas import tpu as pltpu

NSAMPLE = 16
LEAKY = 0.1
TILE_N = 128


def _leaky(v):
    return jnp.where(v >= 0, v, LEAKY * v)


def _mlp_block(feat_ref, w0t_ref, w1t_ref, wn0t_ref, wnb_ref, wlin2t_ref, out_ref):
    rows = feat_ref[0]
    R = rows.shape[0]
    z0 = _leaky(jax.lax.dot(rows, w0t_ref[...],
                            preferred_element_type=jnp.float32))
    z1 = _leaky(jax.lax.dot(z0, w1t_ref[...],
                            preferred_element_type=jnp.float32))
    d = rows[:, 128:131]
    w0 = jax.nn.relu(jax.lax.dot(d, wn0t_ref[...][0:3, 0:8],
                                 preferred_element_type=jnp.float32)
                     + wnb_ref[...][0:1, 0:8])
    w1 = jax.nn.relu(jax.lax.dot(w0, wn0t_ref[...][8:16, 0:8],
                                 preferred_element_type=jnp.float32)
                     + wnb_ref[...][1:2, 0:8])
    w2 = jax.nn.relu(jax.lax.dot(w1, wn0t_ref[...][16:24, 0:8],
                                 preferred_element_type=jnp.float32)
                     + wnb_ref[...][2:3, 0:8])

    w2t = w2.T
    sub_i = jax.lax.broadcasted_iota(jnp.int32, (128, 256), 0)
    lane_i = jax.lax.broadcasted_iota(jnp.int32, (128, 256), 1)
    bd_mask = (sub_i // 8) == (lane_i // 16)
    n_groups = R // 256
    s_parts = []
    for g in range(n_groups):
        wg = w2t[:, g * 256:(g + 1) * 256]
        wg_tiled = jnp.concatenate([wg] * 16, axis=0)
        lhs = jnp.where(bd_mask, wg_tiled, 0.0)
        s_parts.append(jax.lax.dot(lhs, z1[g * 256:(g + 1) * 256, :],
                                   preferred_element_type=jnp.float32))
    s_all = jnp.concatenate(s_parts, axis=0)
    s_r = s_all.reshape(R // 16, 1024)
    out = _leaky(jax.lax.dot(s_r, wlin2t_ref[...],
                             preferred_element_type=jnp.float32))
    out_ref[0] = out.T


def _fused_mlp(feat, w0t, w1t, wn0t, wnb, wlin2t, B, N1):
    K = NSAMPLE
    grid = (B, N1 // TILE_N)
    return pl.pallas_call(
        _mlp_block,
        grid=grid,
        in_specs=[
            pl.BlockSpec((1, TILE_N * K, feat.shape[-1]),
                         lambda b, n: (b, n, 0)),
            pl.BlockSpec((feat.shape[-1], 128), lambda b, n: (0, 0)),
            pl.BlockSpec((128, 128), lambda b, n: (0, 0)),
            pl.BlockSpec((24, 8), lambda b, n: (0, 0)),
            pl.BlockSpec((3, 8), lambda b, n: (0, 0)),
            pl.BlockSpec((1024, 128), lambda b, n: (0, 0)),
        ],
        out_specs=pl.BlockSpec((1, 128, TILE_N), lambda b, n: (b, 0, n)),
        out_shape=jax.ShapeDtypeStruct((B, 128, N1), jnp.float32),
    )(feat, w0t, w1t, wn0t, wnb, wlin2t)


def kernel(xyz1, xyz2, points2, W0, W1, wn_w0, wn_b0, wn_w1, wn_b1, wn_w2,
           wn_b2, W_lin):
    B, _, N1 = xyz1.shape
    N2 = xyz2.shape[2]
    C = points2.shape[1]
    K = NSAMPLE

    x1 = jnp.transpose(xyz1, (0, 2, 1))
    x2 = jnp.transpose(xyz2, (0, 2, 1))
    p2 = jnp.transpose(points2, (0, 2, 1))

    sqr = -2.0 * jnp.einsum('bnc,bmc->bnm', x1, x2)
    sqr = sqr + jnp.sum(x1 ** 2, axis=-1)[:, :, None]
    sqr = sqr + jnp.sum(x2 ** 2, axis=-1)[:, None, :]
    _, knn_idx = jax.lax.top_k(-sqr, K)

    gather = jax.vmap(lambda pts, ind: pts[ind])
    neighbor_xyz = gather(x2, knn_idx)
    direction_xyz = neighbor_xyz - x1[:, :, None, :]
    grouped_points2 = gather(p2, knn_idx)
    feat = jnp.concatenate([grouped_points2, direction_xyz], axis=-1)
    feat = feat.reshape(B, N1 * K, C + 3)

    w0t = W0.T
    w1t = W1.T
    wn0t = jnp.concatenate([
        jnp.pad(wn_w0.T, ((0, 5), (0, 0))),
        wn_w1.T, wn_w2.T], axis=0)
    wnb = jnp.stack([wn_b0, wn_b1, wn_b2], axis=0)
    wlin2t = W_lin.reshape(128, 128, 8).transpose(2, 1, 0).reshape(1024, 128)

    return _fused_mlp(feat, w0t, w1t, wn0t, wnb, wlin2t, B, N1)
